# Initial kernel scaffold; baseline (speedup 1.0000x reference)
import functools

import jax
import jax.numpy as jnp
from jax import lax
from jax.experimental import pallas as pl
from jax.experimental.pallas import tpu as pltpu
from jax.experimental.pallas import tpu_sc as plsc

B = 16384
ID_DIM = 64
FEAT_DIM = 32
MAX_LEN = 20
NC = 2
NS = 16
NW = NC * NS
NQ = B // NW
IC = 128
TQ = 32
NTC = NQ // TQ


def _body(qid_hbm, idtab_hbm, cattab_hbm, brandtab_hbm, tagstab_hbm,
          bufcat_hbm, bufbrand_hbm, buftags_hbm, out_hbm,
          qid_v, cat_idx, brand_idx, flat_pos, tags_if,
          id_rows, cat_rows, brand_rows, tbuf, outtag, sem, sem_w):
  wid = lax.axis_index("s") * NC + lax.axis_index("c")
  base = wid * NQ

  pltpu.sync_copy(qid_hbm.at[pl.ds(base, NQ)], qid_v)

  def fire_idx(j, c):
    pltpu.make_async_copy(
        bufcat_hbm.at[qid_v.at[pl.ds(j * IC, IC)]],
        cat_idx.at[pl.ds(j * IC, IC)], sem).start()
    pltpu.make_async_copy(
        bufbrand_hbm.at[qid_v.at[pl.ds(j * IC, IC)]],
        brand_idx.at[pl.ds(j * IC, IC)], sem).start()
    return c
  lax.fori_loop(0, NQ // IC, fire_idx, 0)

  def fire_id(j, c):
    pltpu.make_async_copy(
        idtab_hbm.at[qid_v.at[pl.ds(j * IC, IC)]],
        id_rows.at[pl.ds(j * IC, IC)], sem).start()
    return c
  lax.fori_loop(0, NQ // IC, fire_id, 0)

  def fp_body(g, c):
    q20 = qid_v[pl.ds(g * 16, 16)] * MAX_LEN
    for s in range(MAX_LEN):
      flat_pos[pl.ds(s * NQ + g * 16, 16)] = q20 + s
    return c
  lax.fori_loop(0, NQ // 16, fp_body, 0)

  def wait_small(j, c):
    pltpu.make_async_copy(
        bufcat_hbm.at[qid_v.at[pl.ds(0, IC)]],
        cat_idx.at[pl.ds(0, IC)], sem).wait()
    return c
  lax.fori_loop(0, 2 * (NQ // IC), wait_small, 0)

  def fire_tif(k, c):
    pltpu.make_async_copy(
        buftags_hbm.at[flat_pos.at[pl.ds(k * IC, IC)]],
        tags_if.at[pl.ds(k * IC, IC)], sem).start()
    return c
  lax.fori_loop(0, (NQ * MAX_LEN) // IC, fire_tif, 0)

  def fire_feat(j, c):
    pltpu.make_async_copy(
        cattab_hbm.at[cat_idx.at[pl.ds(j * IC, IC)]],
        cat_rows.at[pl.ds(j * IC, IC)], sem).start()
    pltpu.make_async_copy(
        brandtab_hbm.at[brand_idx.at[pl.ds(j * IC, IC)]],
        brand_rows.at[pl.ds(j * IC, IC)], sem).start()
    return c
  lax.fori_loop(0, NQ // IC, fire_feat, 0)

  def wait_id(j, c):
    pltpu.make_async_copy(
        idtab_hbm.at[qid_v.at[pl.ds(0, IC)]],
        id_rows.at[pl.ds(0, IC)], sem).wait()
    return c
  lax.fori_loop(0, NQ // IC, wait_id, 0)

  def wait_tif(k, c):
    pltpu.make_async_copy(
        buftags_hbm.at[flat_pos.at[pl.ds(0, IC)]],
        tags_if.at[pl.ds(0, IC)], sem).wait()
    return c
  lax.fori_loop(0, (NQ * MAX_LEN) // IC, wait_tif, 0)

  def wait_feat(j, c):
    pltpu.make_async_copy(
        cattab_hbm.at[cat_idx.at[pl.ds(0, IC)]],
        cat_rows.at[pl.ds(0, IC)], sem).wait()
    return c
  lax.fori_loop(0, 2 * (NQ // IC), wait_feat, 0)

  pltpu.make_async_copy(
      id_rows, out_hbm.at[pl.ds(base, NQ), pl.ds(0, ID_DIM)], sem_w).start()
  pltpu.make_async_copy(
      cat_rows, out_hbm.at[pl.ds(base, NQ), pl.ds(ID_DIM, FEAT_DIM)],
      sem_w).start()
  pltpu.make_async_copy(
      brand_rows,
      out_hbm.at[pl.ds(base, NQ), pl.ds(ID_DIM + FEAT_DIM, FEAT_DIM)],
      sem_w).start()

  def tag_chunk(c, carry):
    def fire_s(s, cc):
      pltpu.make_async_copy(
          tagstab_hbm.at[tags_if.at[pl.ds(s * NQ + c * TQ, TQ)]],
          tbuf.at[s], sem).start()
      return cc
    lax.fori_loop(0, MAX_LEN, fire_s, 0)

    def wait_s(s, cc):
      pltpu.make_async_copy(
          tagstab_hbm.at[tags_if.at[pl.ds(0, TQ)]],
          tbuf.at[0], sem).wait()
      return cc
    lax.fori_loop(0, MAX_LEN, wait_s, 0)

    def red(q, cc):
      a0 = tbuf[0, q, pl.ds(0, 16)]
      a1 = tbuf[0, q, pl.ds(16, 16)]
      for s in range(1, MAX_LEN):
        a0 = a0 + tbuf[s, q, pl.ds(0, 16)]
        a1 = a1 + tbuf[s, q, pl.ds(16, 16)]
      outtag[q, pl.ds(0, 16)] = a0
      outtag[q, pl.ds(16, 16)] = a1
      return cc
    lax.fori_loop(0, TQ, red, 0)

    pltpu.sync_copy(
        outtag,
        out_hbm.at[pl.ds(base + c * TQ, TQ),
                   pl.ds(ID_DIM + 2 * FEAT_DIM, FEAT_DIM)])
    return carry
  lax.fori_loop(0, NTC, tag_chunk, 0)

  pltpu.make_async_copy(
      id_rows, out_hbm.at[pl.ds(base, NQ), pl.ds(0, ID_DIM)], sem_w).wait()
  pltpu.make_async_copy(
      cat_rows, out_hbm.at[pl.ds(base, NQ), pl.ds(ID_DIM, FEAT_DIM)],
      sem_w).wait()
  pltpu.make_async_copy(
      brand_rows,
      out_hbm.at[pl.ds(base, NQ), pl.ds(ID_DIM + FEAT_DIM, FEAT_DIM)],
      sem_w).wait()


@jax.jit
def _run(query_ids, id_table, cat_table, brand_table, tags_table,
         buf_category, buf_brand, buf_tags_flat):
  mesh = plsc.VectorSubcoreMesh(core_axis_name="c", subcore_axis_name="s")
  return pl.kernel(
      _body,
      out_type=jax.ShapeDtypeStruct((B, ID_DIM + 3 * FEAT_DIM), jnp.float32),
      mesh=mesh,
      scratch_types=[
          pltpu.VMEM((NQ,), jnp.int32),
          pltpu.VMEM((NQ,), jnp.int32),
          pltpu.VMEM((NQ,), jnp.int32),
          pltpu.VMEM((NQ * MAX_LEN,), jnp.int32),
          pltpu.VMEM((NQ * MAX_LEN,), jnp.int32),
          pltpu.VMEM((NQ, ID_DIM), jnp.float32),
          pltpu.VMEM((NQ, FEAT_DIM), jnp.float32),
          pltpu.VMEM((NQ, FEAT_DIM), jnp.float32),
          pltpu.VMEM((MAX_LEN, TQ, FEAT_DIM), jnp.float32),
          pltpu.VMEM((TQ, FEAT_DIM), jnp.float32),
          pltpu.SemaphoreType.DMA,
          pltpu.SemaphoreType.DMA,
      ],
  )(query_ids, id_table, cat_table, brand_table, tags_table,
    buf_category, buf_brand, buf_tags_flat)


def kernel(query_ids, id_table, cat_table, brand_table, tags_table,
           buf_category, buf_brand, buf_tags):
  return _run(query_ids.astype(jnp.int32), id_table, cat_table, brand_table,
              tags_table, buf_category.astype(jnp.int32),
              buf_brand.astype(jnp.int32),
              buf_tags.astype(jnp.int32).reshape(-1))


# speedup vs baseline: 1.3513x; 1.3513x over previous
import functools

import jax
import jax.numpy as jnp
from jax import lax
from jax.experimental import pallas as pl
from jax.experimental.pallas import tpu as pltpu
from jax.experimental.pallas import tpu_sc as plsc

B = 16384
ID_DIM = 64
FEAT_DIM = 32
MAX_LEN = 20
NC = 2
NS = 16
NW = NC * NS
NQ = B // NW
IC = 128
TQ = 32
NTC = NQ // TQ


def _body(qid_hbm, idtab_hbm, cattab_hbm, brandtab_hbm, tagstab_hbm,
          bufcat_hbm, bufbrand_hbm, buftags_hbm, out_hbm,
          qid_v, cat_idx, brand_idx, flat_pos, tags_if,
          id_rows, cat_rows, brand_rows, tbuf, outtag, sem, sem_w):
  wid = lax.axis_index("s") * NC + lax.axis_index("c")
  base = wid * NQ

  pltpu.sync_copy(qid_hbm.at[pl.ds(base, NQ)], qid_v)

  def fire_idx(j, c):
    pltpu.make_async_copy(
        bufcat_hbm.at[qid_v.at[pl.ds(j * IC, IC)]],
        cat_idx.at[pl.ds(j * IC, IC)], sem).start()
    pltpu.make_async_copy(
        bufbrand_hbm.at[qid_v.at[pl.ds(j * IC, IC)]],
        brand_idx.at[pl.ds(j * IC, IC)], sem).start()
    return c
  lax.fori_loop(0, NQ // IC, fire_idx, 0)

  def fire_id(j, c):
    pltpu.make_async_copy(
        idtab_hbm.at[qid_v.at[pl.ds(j * IC, IC)]],
        id_rows.at[pl.ds(j * IC, IC)], sem).start()
    return c
  lax.fori_loop(0, NQ // IC, fire_id, 0)

  def fp_body(g, c):
    q20 = qid_v[pl.ds(g * 16, 16)] * MAX_LEN
    for s in range(MAX_LEN):
      flat_pos[pl.ds(s * NQ + g * 16, 16)] = q20 + s
    return c
  lax.fori_loop(0, NQ // 16, fp_body, 0)

  def wait_small(j, c):
    pltpu.make_async_copy(
        bufcat_hbm.at[qid_v.at[pl.ds(0, IC)]],
        cat_idx.at[pl.ds(0, IC)], sem).wait()
    return c
  lax.fori_loop(0, 2 * (NQ // IC), wait_small, 0)

  def fire_tif(k, c):
    pltpu.make_async_copy(
        buftags_hbm.at[flat_pos.at[pl.ds(k * IC, IC)]],
        tags_if.at[pl.ds(k * IC, IC)], sem).start()
    return c
  lax.fori_loop(0, (NQ * MAX_LEN) // IC, fire_tif, 0)

  def fire_feat(j, c):
    pltpu.make_async_copy(
        cattab_hbm.at[cat_idx.at[pl.ds(j * IC, IC)]],
        cat_rows.at[pl.ds(j * IC, IC)], sem).start()
    pltpu.make_async_copy(
        brandtab_hbm.at[brand_idx.at[pl.ds(j * IC, IC)]],
        brand_rows.at[pl.ds(j * IC, IC)], sem).start()
    return c
  lax.fori_loop(0, NQ // IC, fire_feat, 0)

  def wait_id(j, c):
    pltpu.make_async_copy(
        idtab_hbm.at[qid_v.at[pl.ds(0, IC)]],
        id_rows.at[pl.ds(0, IC)], sem).wait()
    return c
  lax.fori_loop(0, NQ // IC, wait_id, 0)

  def wait_tif(k, c):
    pltpu.make_async_copy(
        buftags_hbm.at[flat_pos.at[pl.ds(0, IC)]],
        tags_if.at[pl.ds(0, IC)], sem).wait()
    return c
  lax.fori_loop(0, (NQ * MAX_LEN) // IC, wait_tif, 0)

  def wait_feat(j, c):
    pltpu.make_async_copy(
        cattab_hbm.at[cat_idx.at[pl.ds(0, IC)]],
        cat_rows.at[pl.ds(0, IC)], sem).wait()
    return c
  lax.fori_loop(0, 2 * (NQ // IC), wait_feat, 0)

  pltpu.make_async_copy(
      id_rows, out_hbm.at[pl.ds(base, NQ), pl.ds(0, ID_DIM)], sem_w).start()
  pltpu.make_async_copy(
      cat_rows, out_hbm.at[pl.ds(base, NQ), pl.ds(ID_DIM, FEAT_DIM)],
      sem_w).start()
  pltpu.make_async_copy(
      brand_rows,
      out_hbm.at[pl.ds(base, NQ), pl.ds(ID_DIM + FEAT_DIM, FEAT_DIM)],
      sem_w).start()

  def tag_chunk(c, carry):
    def fire_s(s, cc):
      pltpu.make_async_copy(
          tagstab_hbm.at[tags_if.at[pl.ds(s * NQ + c * TQ, TQ)]],
          tbuf.at[s], sem).start()
      return cc
    lax.fori_loop(0, MAX_LEN, fire_s, 0)

    def wait_s(s, cc):
      pltpu.make_async_copy(
          tagstab_hbm.at[tags_if.at[pl.ds(0, TQ)]],
          tbuf.at[0], sem).wait()
      return cc
    lax.fori_loop(0, MAX_LEN, wait_s, 0)

    def red(q, cc):
      a0 = tbuf[0, q, pl.ds(0, 16)]
      a1 = tbuf[0, q, pl.ds(16, 16)]
      for s in range(1, MAX_LEN):
        a0 = a0 + tbuf[s, q, pl.ds(0, 16)]
        a1 = a1 + tbuf[s, q, pl.ds(16, 16)]
      outtag[q, pl.ds(0, 16)] = a0
      outtag[q, pl.ds(16, 16)] = a1
      return cc
    lax.fori_loop(0, TQ, red, 0)

    pltpu.sync_copy(
        outtag,
        out_hbm.at[pl.ds(base + c * TQ, TQ),
                   pl.ds(ID_DIM + 2 * FEAT_DIM, FEAT_DIM)])
    return carry
  lax.fori_loop(0, NTC, tag_chunk, 0)

  pltpu.make_async_copy(
      id_rows, out_hbm.at[pl.ds(base, NQ), pl.ds(0, ID_DIM)], sem_w).wait()
  pltpu.make_async_copy(
      cat_rows, out_hbm.at[pl.ds(base, NQ), pl.ds(ID_DIM, FEAT_DIM)],
      sem_w).wait()
  pltpu.make_async_copy(
      brand_rows,
      out_hbm.at[pl.ds(base, NQ), pl.ds(ID_DIM + FEAT_DIM, FEAT_DIM)],
      sem_w).wait()


@jax.jit
def _run(query_ids, id_table, cat_table, brand_table, tags_table,
         buf_category, buf_brand, buf_tags_flat):
  mesh = plsc.VectorSubcoreMesh(core_axis_name="c", subcore_axis_name="s")
  return pl.kernel(
      _body,
      out_type=jax.ShapeDtypeStruct((B, ID_DIM + 3 * FEAT_DIM), jnp.float32),
      mesh=mesh,
      compiler_params=pltpu.CompilerParams(use_tc_tiling_on_sc=False),
      scratch_types=[
          pltpu.VMEM((NQ,), jnp.int32),
          pltpu.VMEM((NQ,), jnp.int32),
          pltpu.VMEM((NQ,), jnp.int32),
          pltpu.VMEM((NQ * MAX_LEN,), jnp.int32),
          pltpu.VMEM((NQ * MAX_LEN,), jnp.int32),
          pltpu.VMEM((NQ, ID_DIM), jnp.float32),
          pltpu.VMEM((NQ, FEAT_DIM), jnp.float32),
          pltpu.VMEM((NQ, FEAT_DIM), jnp.float32),
          pltpu.VMEM((MAX_LEN, TQ, FEAT_DIM), jnp.float32),
          pltpu.VMEM((TQ, FEAT_DIM), jnp.float32),
          pltpu.SemaphoreType.DMA,
          pltpu.SemaphoreType.DMA,
      ],
  )(query_ids, id_table, cat_table, brand_table, tags_table,
    buf_category, buf_brand, buf_tags_flat)


def kernel(query_ids, id_table, cat_table, brand_table, tags_table,
           buf_category, buf_brand, buf_tags):
  return _run(query_ids.astype(jnp.int32), id_table, cat_table, brand_table,
              tags_table, buf_category.astype(jnp.int32),
              buf_brand.astype(jnp.int32),
              buf_tags.astype(jnp.int32).reshape(-1))


# speedup vs baseline: 2.2557x; 1.6693x over previous
import jax
import jax.numpy as jnp
from jax import lax
from jax.experimental import pallas as pl
from jax.experimental.pallas import tpu as pltpu
from jax.experimental.pallas import tpu_sc as plsc

B = 16384
ID_DIM = 64
FEAT_DIM = 32
MAX_LEN = 20
NC = 2
NS = 16
NW = NC * NS
NQ = B // NW
IC = 128
RING = 8
TQ = 32
NTC = NQ // TQ


def _stage1(qid_hbm, idtab_hbm, buftags_hbm, bufcat_hbm, bufbrand_hbm,
            oid_hbm, otif_hbm, ocat_hbm, obrand_hbm,
            qid_v, cat_idx, brand_idx, id_out, tags_if,
            id8, tb8, sem, sem_q0, sem_q1, sem_w):
  wid = lax.axis_index("s") * NC + lax.axis_index("c")
  base = wid * NQ

  pltpu.sync_copy(qid_hbm.at[pl.ds(base, NQ)], qid_v)

  def fire_ent(j, c):
    pltpu.make_async_copy(
        bufcat_hbm.at[qid_v.at[pl.ds(j * IC, IC)]],
        cat_idx.at[pl.ds(j * IC, IC)], sem).start()
    pltpu.make_async_copy(
        bufbrand_hbm.at[qid_v.at[pl.ds(j * IC, IC)]],
        brand_idx.at[pl.ds(j * IC, IC)], sem).start()
    return c
  lax.fori_loop(0, NQ // IC, fire_ent, 0)

  def fire_group(goff, par, sem_q):
    qv = qid_v[pl.ds(goff, 16)]
    for j in range(16):
      r = qv[j]
      rb = pl.multiple_of(r - lax.bitwise_and(r, 7), 8)
      pltpu.make_async_copy(
          idtab_hbm.at[pl.ds(rb, 8)], id8.at[par * 16 + j], sem_q).start()
      pltpu.make_async_copy(
          buftags_hbm.at[pl.ds(rb, 8)], tb8.at[par * 16 + j], sem_q).start()

  def drain_extract(goff, par, phase, sem_q):
    for j in range(16):
      pltpu.make_async_copy(
          idtab_hbm.at[pl.ds(0, 8)], id8.at[par * 16 + j], sem_q).wait()
      pltpu.make_async_copy(
          buftags_hbm.at[pl.ds(0, 8)], tb8.at[par * 16 + j], sem_q).wait()
    qv = qid_v[pl.ds(goff, 16)]
    for j in range(16):
      slot = par * 16 + j
      sub = lax.bitwise_and(qv[j], 7)
      for k in range(ID_DIM // 16):
        id_out[phase, par * 16 + j, pl.ds(k * 16, 16)] = (
            id8[slot, sub, pl.ds(k * 16, 16)])
      tags_if[pl.ds((goff + j) * MAX_LEN, 16)] = tb8[slot, sub, pl.ds(0, 16)]
      tags_if[pl.ds((goff + j) * MAX_LEN + 4, 16)] = tb8[slot, sub,
                                                        pl.ds(4, 16)]

  fire_group(0, 0, sem_q0)
  fire_group(16, 1, sem_q1)

  def pair_body(gg, c):
    goff = gg * 32
    phase = lax.bitwise_and(gg, 1)

    @pl.when(gg >= 2)
    def _():
      pltpu.make_async_copy(
          id_out.at[0], oid_hbm.at[pl.ds(base, 32)], sem_w).wait()

    drain_extract(goff, 0, phase, sem_q0)

    @pl.when(gg < NQ // 32 - 1)
    def _():
      fire_group(goff + 32, 0, sem_q0)
    drain_extract(goff + 16, 1, phase, sem_q1)

    @pl.when(gg < NQ // 32 - 1)
    def _():
      fire_group(goff + 48, 1, sem_q1)

    pltpu.make_async_copy(
        id_out.at[phase], oid_hbm.at[pl.ds(base + goff, 32)], sem_w).start()
    return c
  lax.fori_loop(0, NQ // 32, pair_body, 0)

  for _ in range(2):
    pltpu.make_async_copy(
        id_out.at[0], oid_hbm.at[pl.ds(base, 32)], sem_w).wait()

  def wait_ent(j, c):
    pltpu.make_async_copy(
        bufcat_hbm.at[qid_v.at[pl.ds(0, IC)]],
        cat_idx.at[pl.ds(0, IC)], sem).wait()
    pltpu.make_async_copy(
        bufbrand_hbm.at[qid_v.at[pl.ds(0, IC)]],
        brand_idx.at[pl.ds(0, IC)], sem).wait()
    return c
  lax.fori_loop(0, NQ // IC, wait_ent, 0)

  w1 = pltpu.make_async_copy(
      tags_if, otif_hbm.at[pl.ds(base * MAX_LEN, NQ * MAX_LEN)], sem_w)
  w2 = pltpu.make_async_copy(cat_idx, ocat_hbm.at[pl.ds(base, NQ)], sem_w)
  w3 = pltpu.make_async_copy(brand_idx, obrand_hbm.at[pl.ds(base, NQ)], sem_w)
  w1.start(), w2.start(), w3.start()
  w1.wait(), w2.wait(), w3.wait()


def _stage2(catidx_hbm, brandidx_hbm, tif_hbm,
            cattab_hbm, brandtab_hbm, tagstab_hbm,
            ocat_hbm, obrand_hbm, otags_hbm,
            cat_idx, brand_idx, tif_v, cat_rows, brand_rows, tags_acc,
            tchunk, sem, sem_t, sem_w):
  wid = lax.axis_index("s") * NC + lax.axis_index("c")
  base = wid * NQ

  pltpu.sync_copy(catidx_hbm.at[pl.ds(base, NQ)], cat_idx)
  pltpu.sync_copy(brandidx_hbm.at[pl.ds(base, NQ)], brand_idx)
  pltpu.sync_copy(
      tif_hbm.at[pl.ds(base * MAX_LEN, NQ * MAX_LEN)], tif_v)

  def fire_feat(j, c):
    pltpu.make_async_copy(
        cattab_hbm.at[cat_idx.at[pl.ds(j * IC, IC)]],
        cat_rows.at[pl.ds(j * IC, IC)], sem).start()
    pltpu.make_async_copy(
        brandtab_hbm.at[brand_idx.at[pl.ds(j * IC, IC)]],
        brand_rows.at[pl.ds(j * IC, IC)], sem).start()
    return c
  lax.fori_loop(0, NQ // IC, fire_feat, 0)

  NB = (TQ * MAX_LEN) // IC

  def fire_chunk(c, buf):
    def fire_k(k, cc):
      pltpu.make_async_copy(
          tagstab_hbm.at[tif_v.at[pl.ds(c * TQ * MAX_LEN + k * IC, IC)]],
          tchunk.at[buf, pl.ds(k * IC, IC)], sem_t).start()
      return cc
    lax.fori_loop(0, NB, fire_k, 0)

  def wait_chunk():
    def wait_k(k, cc):
      pltpu.make_async_copy(
          tagstab_hbm.at[tif_v.at[pl.ds(0, IC)]],
          tchunk.at[0, pl.ds(0, IC)], sem_t).wait()
      return cc
    lax.fori_loop(0, NB, wait_k, 0)

  fire_chunk(0, 0)

  def chunk_body(c, carry):
    buf = lax.bitwise_and(c, 1)

    @pl.when(c < NTC - 1)
    def _():
      fire_chunk(c + 1, 1 - buf)
    wait_chunk()

    def red(q, cc):
      a0 = tchunk[buf, q * MAX_LEN, pl.ds(0, 16)]
      a1 = tchunk[buf, q * MAX_LEN, pl.ds(16, 16)]
      for s in range(1, MAX_LEN):
        a0 = a0 + tchunk[buf, q * MAX_LEN + s, pl.ds(0, 16)]
        a1 = a1 + tchunk[buf, q * MAX_LEN + s, pl.ds(16, 16)]
      tags_acc[c * TQ + q, pl.ds(0, 16)] = a0
      tags_acc[c * TQ + q, pl.ds(16, 16)] = a1
      return cc
    lax.fori_loop(0, TQ, red, 0)
    return carry
  lax.fori_loop(0, NTC, chunk_body, 0)

  def wait_feat(j, c):
    pltpu.make_async_copy(
        cattab_hbm.at[cat_idx.at[pl.ds(0, IC)]],
        cat_rows.at[pl.ds(0, IC)], sem).wait()
    pltpu.make_async_copy(
        brandtab_hbm.at[brand_idx.at[pl.ds(0, IC)]],
        brand_rows.at[pl.ds(0, IC)], sem).wait()
    return c
  lax.fori_loop(0, NQ // IC, wait_feat, 0)

  w0 = pltpu.make_async_copy(cat_rows, ocat_hbm.at[pl.ds(base, NQ)], sem_w)
  w1 = pltpu.make_async_copy(brand_rows, obrand_hbm.at[pl.ds(base, NQ)], sem_w)
  w2 = pltpu.make_async_copy(tags_acc, otags_hbm.at[pl.ds(base, NQ)], sem_w)
  w0.start(), w1.start(), w2.start()
  w0.wait(), w1.wait(), w2.wait()


@jax.jit
def _run(query_ids, id_table, cat_table, brand_table, tags_table,
         buf_category, buf_brand, buf_tags):
  mesh = plsc.VectorSubcoreMesh(core_axis_name="c", subcore_axis_name="s")
  id_emb, tags_if, cat_idx, brand_idx = pl.kernel(
      _stage1,
      out_type=(
          jax.ShapeDtypeStruct((B, ID_DIM), jnp.float32),
          jax.ShapeDtypeStruct((B * MAX_LEN,), jnp.int32),
          jax.ShapeDtypeStruct((B,), jnp.int32),
          jax.ShapeDtypeStruct((B,), jnp.int32),
      ),
      mesh=mesh,
      scratch_types=[
          pltpu.VMEM((NQ,), jnp.int32),
          pltpu.VMEM((NQ,), jnp.int32),
          pltpu.VMEM((NQ,), jnp.int32),
          pltpu.VMEM((2, 32, ID_DIM), jnp.float32),
          pltpu.VMEM((NQ * MAX_LEN,), jnp.int32),
          pltpu.VMEM((32, 8, ID_DIM), jnp.float32),
          pltpu.VMEM((32, 8, MAX_LEN), jnp.int32),
          pltpu.SemaphoreType.DMA,
          pltpu.SemaphoreType.DMA,
          pltpu.SemaphoreType.DMA,
          pltpu.SemaphoreType.DMA,
      ],
  )(query_ids, id_table, buf_tags, buf_category, buf_brand)

  ocat, obrand, otags = pl.kernel(
      _stage2,
      out_type=(
          jax.ShapeDtypeStruct((B, FEAT_DIM), jnp.float32),
          jax.ShapeDtypeStruct((B, FEAT_DIM), jnp.float32),
          jax.ShapeDtypeStruct((B, FEAT_DIM), jnp.float32),
      ),
      mesh=mesh,
      compiler_params=pltpu.CompilerParams(use_tc_tiling_on_sc=False),
      scratch_types=[
          pltpu.VMEM((NQ,), jnp.int32),
          pltpu.VMEM((NQ,), jnp.int32),
          pltpu.VMEM((NQ * MAX_LEN,), jnp.int32),
          pltpu.VMEM((NQ, FEAT_DIM), jnp.float32),
          pltpu.VMEM((NQ, FEAT_DIM), jnp.float32),
          pltpu.VMEM((NQ, FEAT_DIM), jnp.float32),
          pltpu.VMEM((2, TQ * MAX_LEN, FEAT_DIM), jnp.float32),
          pltpu.SemaphoreType.DMA,
          pltpu.SemaphoreType.DMA,
          pltpu.SemaphoreType.DMA,
      ],
  )(cat_idx, brand_idx, tags_if, cat_table, brand_table, tags_table)

  return jnp.concatenate([id_emb, ocat, obrand, otags], axis=-1)


def kernel(query_ids, id_table, cat_table, brand_table, tags_table,
           buf_category, buf_brand, buf_tags):
  return _run(query_ids.astype(jnp.int32), id_table, cat_table, brand_table,
              tags_table, buf_category.astype(jnp.int32),
              buf_brand.astype(jnp.int32), buf_tags.astype(jnp.int32))
